# Initial kernel scaffold; baseline (speedup 1.0000x reference)
import jax
import jax.numpy as jnp
from jax.experimental import pallas as pl


def kernel(random_vector, global_pre, global_post, proj, msg, update):
    raise NotImplementedError("write your pallas kernel here")


# speedup vs baseline: 36.2050x; 36.2050x over previous
import numpy as np
import jax
import jax.numpy as jnp
from jax.experimental import pallas as pl
from jax.experimental.pallas import tpu as pltpu

_NF = 64
_NG = 32
_NB = 4
_NS = 6
_B = 4
_LVL = [_NB ** i for i in range(_NS + 1)]
_OFF = [int(v) for v in np.cumsum([0] + _LVL[:-1])]
_NN = sum(_LVL)


def _relu_mm(x, W, b):
    return jnp.maximum(jnp.dot(x, W, preferred_element_type=jnp.float32) + b, 0.0)


def _mlp_chain(x, layers):
    for W, b in layers:
        x = _relu_mm(x, W, b)
    return x


def _bcast_g(g, L):
    if L == 1:
        return g
    return jnp.concatenate(
        [jnp.broadcast_to(g[b:b + 1, :], (L, _NG)) for b in range(_B)], axis=0)


def _forward_kernel(rv_ref, *refs):
    refs = list(refs)
    rep_ref = refs.pop()
    out_ref = refs.pop()

    def take(n):
        nonlocal refs
        layers = []
        for _ in range(n):
            W = refs.pop(0)[...]
            b = refs.pop(0)[...]
            layers.append((W, b))
        return layers

    gpre = take(2)
    gpost = take(2)
    wproj = take(3)
    wmsg = take(3)
    wupd = take(3)

    xs = [rv_ref[...].reshape(_B, _NF)]

    for k in range(1, _NS + 1):
        gsum = None
        for l in range(k):
            h = _mlp_chain(xs[l], gpre)
            L = _LVL[l]
            if L == 1:
                s = h
            else:
                s = jnp.concatenate(
                    [jnp.sum(h[b * L:(b + 1) * L, :], axis=0, keepdims=True)
                     for b in range(_B)], axis=0)
            gsum = s if gsum is None else gsum + s
        g = _mlp_chain(gsum, gpost)

        leaf = xs[k - 1]
        n_leaf = _B * _LVL[k - 1]
        gleaf = _bcast_g(g, _LVL[k - 1])
        new = _mlp_chain(jnp.concatenate([leaf, gleaf], axis=1), wproj)
        for c in range(_NB):
            rep_ref[pl.Slice(c, n_leaf, _NB), :] = \
                new[:, c * _NF:(c + 1) * _NF]
        xs.append(rep_ref[pl.ds(0, n_leaf * _NB), :])

        Ms = []
        for l in range(k):
            gl = _bcast_g(g, _LVL[l])
            Ms.append(_mlp_chain(jnp.concatenate([xs[l], gl], axis=1), wmsg))

        aggr = jnp.zeros((_B, _NF), dtype=jnp.float32)
        for l in range(k + 1):
            gl = _bcast_g(g, _LVL[l])
            x_old = xs[l]
            xs[l] = _mlp_chain(
                jnp.concatenate([x_old, aggr, gl], axis=1), wupd)
            if l < k:
                a = aggr + Ms[l]
                n = _B * _LVL[l]
                for c in range(_NB):
                    rep_ref[pl.Slice(c, n, _NB), :] = a
                aggr = rep_ref[pl.ds(0, n * _NB), :]

    for b in range(_B):
        for l in range(_NS + 1):
            L = _LVL[l]
            out_ref[pl.ds(b * _NN + _OFF[l], L), :] = xs[l][b * L:(b + 1) * L, :]


def kernel(random_vector, global_pre, global_post, proj, msg, update):
    flat = []
    for layers in (global_pre, global_post, proj, msg, update):
        for W, b in layers:
            flat.append(W)
            flat.append(b.reshape(1, -1))
    out = pl.pallas_call(
        _forward_kernel,
        out_shape=jax.ShapeDtypeStruct((_B * _NN, _NF), jnp.float32),
        scratch_shapes=[pltpu.VMEM((_B * _LVL[_NS], _NF), jnp.float32)],
    )(random_vector.reshape(_B, _NF), *flat)
    return out
